# baseline (speedup 1.0000x reference)
import functools
import numpy as np
import jax
import jax.numpy as jnp
from jax import lax
from jax.experimental import pallas as pl
from jax.experimental.pallas import tpu as pltpu
from jax.experimental.pallas import tpu_sc as plsc

N_LEVELS = 16
F = 2
BASE = 16
TARGET = 2048
LOG2_T = 19
T = 2 ** LOG2_T
DIM = 3
N_PTS = 131072
GRID_OUT = 64
HIDDEN = (GRID_OUT + 15) // 16 * 16
SCALE = np.exp2(np.log2(TARGET / BASE) / (N_LEVELS - 1))
RES = [int(np.floor(BASE * SCALE ** l)) for l in range(N_LEVELS)]
DENSE = [(r + 1) ** 3 <= T for r in RES]
P1, P2 = 2654435761, 805459861
ENC = N_LEVELS * F

NC, NS, LANES = 2, 16, 16
NW = NC * NS
PW = N_PTS // NW
CH = 16
NCHUNK = PW // CH
HBUF = 512
FLUSH_EVERY = HBUF // CH


def _encode_body(xt_hbm, table_hbm, h_hbm, xv, wb, colb, hbuf, sem, *lvl_scratch):
    idxrefs = lvl_scratch[:N_LEVELS]
    rowrefs = lvl_scratch[N_LEVELS:]
    wid = lax.axis_index("s") * NC + lax.axis_index("c")
    base = wid * PW
    pltpu.sync_copy(xt_hbm.at[:, pl.ds(base, PW)], xv)
    iota = lax.iota(jnp.int32, LANES)
    zeros_i = jnp.zeros((LANES,), jnp.int32)
    ones_i = jnp.ones((LANES,), jnp.int32)

    def chunk_body(k, carry):
        off = k * CH
        xc = xv[0, pl.ds(off, CH)]
        yc = xv[1, pl.ds(off, CH)]
        zc = xv[2, pl.ds(off, CH)]

        for l in range(N_LEVELS):
            res = RES[l]
            resf = float(res)
            px = xc * resf
            py = yc * resf
            pz = zc * resf
            ix = px.astype(jnp.int32)
            iy = py.astype(jnp.int32)
            iz = pz.astype(jnp.int32)
            fx = px - ix.astype(jnp.float32)
            fy = py - iy.astype(jnp.float32)
            fz = pz - iz.astype(jnp.float32)
            x0, x1 = ix, jnp.minimum(ix + 1, res)
            y0, y1 = iy, jnp.minimum(iy + 1, res)
            z0, z1 = iz, jnp.minimum(iz + 1, res)
            wx0, wx1 = 1.0 - fx, fx
            wy0, wy1 = 1.0 - fy, fy
            wz0, wz1 = 1.0 - fz, fz
            if not DENSE[l]:
                hx0 = x0.astype(jnp.uint32)
                hx1 = x1.astype(jnp.uint32)
                hy0 = y0.astype(jnp.uint32) * jnp.uint32(P1)
                hy1 = y1.astype(jnp.uint32) * jnp.uint32(P1)
                hz0 = z0.astype(jnp.uint32) * jnp.uint32(P2)
                hz1 = z1.astype(jnp.uint32) * jnp.uint32(P2)
            for c in range(8):
                cxb, cyb, czb = c & 1, (c >> 1) & 1, (c >> 2) & 1
                w = ((wx1 if cxb else wx0) * (wy1 if cyb else wy0)) * (wz1 if czb else wz0)
                if DENSE[l]:
                    s = res + 1
                    idx = (x1 if cxb else x0) + (y1 if cyb else y0) * s \
                        + (z1 if czb else z0) * (s * s) + (l * T)
                else:
                    h = (hx1 if cxb else hx0) ^ (hy1 if cyb else hy0) ^ (hz1 if czb else hz0)
                    idx = (h & jnp.uint32(T - 1)).astype(jnp.int32) + (l * T)
                idxrefs[l][pl.ds(c * LANES, LANES)] = lax.shift_right_logical(idx, 2)
                colb[l, pl.ds(c * LANES, LANES)] = (idx & 3) * 2
                wb[l, pl.ds(c * LANES, LANES)] = w

        copies = []
        for l in range(N_LEVELS):
            cp = pltpu.make_async_copy(
                table_hbm.at[idxrefs[l]], rowrefs[l], sem)
            cp.start()
            copies.append(cp)

        hcol = (k % FLUSH_EVERY) * CH
        for l in range(N_LEVELS):
            copies[l].wait()
            acc0 = jnp.zeros((LANES,), jnp.float32)
            acc1 = jnp.zeros((LANES,), jnp.float32)
            for c in range(8):
                w = wb[l, pl.ds(c * LANES, LANES)]
                colv = colb[l, pl.ds(c * LANES, LANES)]
                rows = c * LANES + iota
                f0 = plsc.load_gather(rowrefs[l], [rows, colv])
                f1 = plsc.load_gather(rowrefs[l], [rows, colv + 1])
                acc0 = acc0 + f0 * w
                acc1 = acc1 + f1 * w
            hbuf[2 * l, pl.ds(hcol, CH)] = acc0
            hbuf[2 * l + 1, pl.ds(hcol, CH)] = acc1

        @pl.when(k % FLUSH_EVERY == FLUSH_EVERY - 1)
        def _():
            ob = pl.multiple_of(base + (k - (FLUSH_EVERY - 1)) * CH, HBUF)
            pltpu.sync_copy(hbuf, h_hbm.at[:, pl.ds(ob, HBUF)])

        return carry

    lax.fori_loop(0, NCHUNK, chunk_body, 0)


@jax.jit
def _encode(xt, table):
    mesh = plsc.VectorSubcoreMesh(core_axis_name="c", subcore_axis_name="s",
                                  num_cores=NC, num_subcores=NS)
    return pl.kernel(
        _encode_body,
        out_type=jax.ShapeDtypeStruct((ENC, N_PTS), jnp.float32),
        mesh=mesh,
        compiler_params=pltpu.CompilerParams(needs_layout_passes=False,
                                             use_tc_tiling_on_sc=False),
        scratch_types=(
            [
                pltpu.VMEM((DIM, PW), jnp.float32),
                pltpu.VMEM((N_LEVELS, 8 * LANES), jnp.float32),
                pltpu.VMEM((N_LEVELS, 8 * LANES), jnp.int32),
                pltpu.VMEM((ENC, HBUF), jnp.float32),
                pltpu.SemaphoreType.DMA,
            ]
            + [pltpu.VMEM((8 * LANES,), jnp.int32) for _ in range(N_LEVELS)]
            + [pltpu.VMEM((8 * LANES, 8), jnp.float32) for _ in range(N_LEVELS)]
        ),
    )(xt, table.reshape(N_LEVELS * T * F // 8, 8))


BLK = 1024


def _mlp_body(h_ref, w0_ref, b0_ref, w1_ref, b1_ref, w2_ref, b2_ref, o_ref):
    h = h_ref[...]
    a = lax.dot_general(h, w0_ref[...], (((0,), (0,)), ((), ())),
                        preferred_element_type=jnp.float32,
                        precision=lax.Precision.HIGHEST)
    a = jnp.maximum(a + b0_ref[...], 0.0)
    a = jnp.dot(a, w1_ref[...], preferred_element_type=jnp.float32,
                precision=lax.Precision.HIGHEST)
    a = jnp.maximum(a + b1_ref[...], 0.0)
    o_ref[...] = jnp.dot(a, w2_ref[...], preferred_element_type=jnp.float32,
                         precision=lax.Precision.HIGHEST) + b2_ref[...]


@jax.jit
def _mlp(h, W0, b0, W1, b1, W2, b2):
    grid = (N_PTS // BLK,)
    return pl.pallas_call(
        _mlp_body,
        grid=grid,
        in_specs=[
            pl.BlockSpec((ENC, BLK), lambda i: (0, i)),
            pl.BlockSpec((ENC, HIDDEN), lambda i: (0, 0)),
            pl.BlockSpec((1, HIDDEN), lambda i: (0, 0)),
            pl.BlockSpec((HIDDEN, HIDDEN), lambda i: (0, 0)),
            pl.BlockSpec((1, HIDDEN), lambda i: (0, 0)),
            pl.BlockSpec((HIDDEN, GRID_OUT), lambda i: (0, 0)),
            pl.BlockSpec((1, GRID_OUT), lambda i: (0, 0)),
        ],
        out_specs=pl.BlockSpec((BLK, GRID_OUT), lambda i: (i, 0)),
        out_shape=jax.ShapeDtypeStruct((N_PTS, GRID_OUT), jnp.float32),
    )(h, W0, b0, W1, b1, W2, b2)


def kernel(x, table, W0, b0, W1, b1, W2, b2):
    xt = x.T
    h = _encode(xt, table)
    return _mlp(h, W0, b0.reshape(1, -1), W1, b1.reshape(1, -1),
                W2, b2.reshape(1, -1))


# speedup vs baseline: 6.3887x; 6.3887x over previous
import functools
import numpy as np
import jax
import jax.numpy as jnp
from jax import lax
from jax.experimental import pallas as pl
from jax.experimental.pallas import tpu as pltpu
from jax.experimental.pallas import tpu_sc as plsc

N_LEVELS = 16
F = 2
BASE = 16
TARGET = 2048
LOG2_T = 19
T = 2 ** LOG2_T
DIM = 3
N_PTS = 131072
GRID_OUT = 64
HIDDEN = (GRID_OUT + 15) // 16 * 16
SCALE = np.exp2(np.log2(TARGET / BASE) / (N_LEVELS - 1))
RES = [int(np.floor(BASE * SCALE ** l)) for l in range(N_LEVELS)]
DENSE = [(r + 1) ** 3 <= T for r in RES]
P1, P2 = 2654435761, 805459861
ENC = N_LEVELS * F

NC, NS, LANES = 2, 16, 16
NW = NC * NS
PW = N_PTS // NW
CH = 16
NCHUNK = PW // CH
HBUF = 512
FLUSH_EVERY = HBUF // CH


def _encode_body(xt_hbm, table_hbm, h_hbm, xv, wb, colb, hbuf, sem, *lvl_scratch):
    idxA = lvl_scratch[:N_LEVELS]
    idxB = lvl_scratch[N_LEVELS:2 * N_LEVELS]
    rowsA = lvl_scratch[2 * N_LEVELS:3 * N_LEVELS]
    rowsB = lvl_scratch[3 * N_LEVELS:]
    wid = lax.axis_index("s") * NC + lax.axis_index("c")
    base = wid * PW
    pltpu.sync_copy(xt_hbm.at[:, pl.ds(base, PW)], xv)
    iota = lax.iota(jnp.int32, LANES)
    zeros_i = jnp.zeros((LANES,), jnp.int32)
    ones_i = jnp.ones((LANES,), jnp.int32)

    def chunk_body(k, carry):
        off = k * CH
        xc = xv[0, pl.ds(off, CH)]
        yc = xv[1, pl.ds(off, CH)]
        zc = xv[2, pl.ds(off, CH)]

        for l in range(N_LEVELS):
            res = RES[l]
            resf = float(res)
            px = xc * resf
            py = yc * resf
            pz = zc * resf
            ix = px.astype(jnp.int32)
            iy = py.astype(jnp.int32)
            iz = pz.astype(jnp.int32)
            fx = px - ix.astype(jnp.float32)
            fy = py - iy.astype(jnp.float32)
            fz = pz - iz.astype(jnp.float32)
            x0, x1 = ix, jnp.minimum(ix + 1, res)
            y0, y1 = iy, jnp.minimum(iy + 1, res)
            z0, z1 = iz, jnp.minimum(iz + 1, res)
            wx0, wx1 = 1.0 - fx, fx
            wy0, wy1 = 1.0 - fy, fy
            wz0, wz1 = 1.0 - fz, fz
            if not DENSE[l]:
                hx0 = x0.astype(jnp.uint32)
                hx1 = x1.astype(jnp.uint32)
                hy0 = y0.astype(jnp.uint32) * jnp.uint32(P1)
                hy1 = y1.astype(jnp.uint32) * jnp.uint32(P1)
                hz0 = z0.astype(jnp.uint32) * jnp.uint32(P2)
                hz1 = z1.astype(jnp.uint32) * jnp.uint32(P2)
            for c in range(8):
                cxb, cyb, czb = c & 1, (c >> 1) & 1, (c >> 2) & 1
                w = ((wx1 if cxb else wx0) * (wy1 if cyb else wy0)) * (wz1 if czb else wz0)
                if DENSE[l]:
                    s = res + 1
                    idx = (x1 if cxb else x0) + (y1 if cyb else y0) * s \
                        + (z1 if czb else z0) * (s * s)
                else:
                    h = (hx1 if cxb else hx0) ^ (hy1 if cyb else hy0) ^ (hz1 if czb else hz0)
                    idx = (h & jnp.uint32(T - 1)).astype(jnp.int32)
                row0 = (l * (T * F // 8)
                        + lax.shift_right_logical(idx, 7) * 32
                        + lax.shift_right_logical(idx & 127, 3))
                idxA[l][pl.ds(c * LANES, LANES)] = row0
                idxB[l][pl.ds(c * LANES, LANES)] = row0 + 16
                colb[l, pl.ds(c * LANES, LANES)] = idx & 7
                wb[l, pl.ds(c * LANES, LANES)] = w

        copies = []
        for l in range(N_LEVELS):
            cpA = pltpu.make_async_copy(table_hbm.at[idxA[l]], rowsA[l], sem)
            cpA.start()
            cpB = pltpu.make_async_copy(table_hbm.at[idxB[l]], rowsB[l], sem)
            cpB.start()
            copies.append((cpA, cpB))

        hcol = (k % FLUSH_EVERY) * CH
        for l in range(N_LEVELS):
            copies[l][0].wait()
            copies[l][1].wait()
            acc0 = jnp.zeros((LANES,), jnp.float32)
            acc1 = jnp.zeros((LANES,), jnp.float32)
            for c in range(8):
                w = wb[l, pl.ds(c * LANES, LANES)]
                colv = colb[l, pl.ds(c * LANES, LANES)]
                rows = c * LANES + iota
                f0 = plsc.load_gather(rowsA[l], [rows, colv])
                f1 = plsc.load_gather(rowsB[l], [rows, colv])
                acc0 = acc0 + f0 * w
                acc1 = acc1 + f1 * w
            hbuf[2 * l, pl.ds(hcol, CH)] = acc0
            hbuf[2 * l + 1, pl.ds(hcol, CH)] = acc1

        @pl.when(k % FLUSH_EVERY == FLUSH_EVERY - 1)
        def _():
            ob = pl.multiple_of(base + (k - (FLUSH_EVERY - 1)) * CH, HBUF)
            pltpu.sync_copy(hbuf, h_hbm.at[:, pl.ds(ob, HBUF)])

        return carry

    lax.fori_loop(0, NCHUNK, chunk_body, 0)


@jax.jit
def _encode(xt, table):
    mesh = plsc.VectorSubcoreMesh(core_axis_name="c", subcore_axis_name="s",
                                  num_cores=NC, num_subcores=NS)
    return pl.kernel(
        _encode_body,
        out_type=jax.ShapeDtypeStruct((ENC, N_PTS), jnp.float32),
        mesh=mesh,
        compiler_params=pltpu.CompilerParams(needs_layout_passes=False,
                                             use_tc_tiling_on_sc=False),
        scratch_types=(
            [
                pltpu.VMEM((DIM, PW), jnp.float32),
                pltpu.VMEM((N_LEVELS, 8 * LANES), jnp.float32),
                pltpu.VMEM((N_LEVELS, 8 * LANES), jnp.int32),
                pltpu.VMEM((ENC, HBUF), jnp.float32),
                pltpu.SemaphoreType.DMA,
            ]
            + [pltpu.VMEM((8 * LANES,), jnp.int32) for _ in range(2 * N_LEVELS)]
            + [pltpu.VMEM((8 * LANES, 8), jnp.float32) for _ in range(2 * N_LEVELS)]
        ),
    )(xt, table)


BLK = 1024


def _mlp_body(h_ref, w0_ref, b0_ref, w1_ref, b1_ref, w2_ref, b2_ref, o_ref):
    h = h_ref[...]
    a = lax.dot_general(h, w0_ref[...], (((0,), (0,)), ((), ())),
                        preferred_element_type=jnp.float32,
                        precision=lax.Precision.HIGHEST)
    a = jnp.maximum(a + b0_ref[...], 0.0)
    a = jnp.dot(a, w1_ref[...], preferred_element_type=jnp.float32,
                precision=lax.Precision.HIGHEST)
    a = jnp.maximum(a + b1_ref[...], 0.0)
    o_ref[...] = jnp.dot(a, w2_ref[...], preferred_element_type=jnp.float32,
                         precision=lax.Precision.HIGHEST) + b2_ref[...]


@jax.jit
def _mlp(h, W0, b0, W1, b1, W2, b2):
    grid = (N_PTS // BLK,)
    return pl.pallas_call(
        _mlp_body,
        grid=grid,
        in_specs=[
            pl.BlockSpec((ENC, BLK), lambda i: (0, i)),
            pl.BlockSpec((ENC, HIDDEN), lambda i: (0, 0)),
            pl.BlockSpec((1, HIDDEN), lambda i: (0, 0)),
            pl.BlockSpec((HIDDEN, HIDDEN), lambda i: (0, 0)),
            pl.BlockSpec((1, HIDDEN), lambda i: (0, 0)),
            pl.BlockSpec((HIDDEN, GRID_OUT), lambda i: (0, 0)),
            pl.BlockSpec((1, GRID_OUT), lambda i: (0, 0)),
        ],
        out_specs=pl.BlockSpec((BLK, GRID_OUT), lambda i: (i, 0)),
        out_shape=jax.ShapeDtypeStruct((N_PTS, GRID_OUT), jnp.float32),
    )(h, W0, b0, W1, b1, W2, b2)


def kernel(x, table, W0, b0, W1, b1, W2, b2):
    xt = x.T
    tphys = (table.transpose(0, 2, 1)
             .reshape(N_LEVELS, F, T // 128, 128)
             .transpose(0, 2, 1, 3)
             .reshape(N_LEVELS * T * F // 8, 8))
    h = _encode(xt, tphys)
    return _mlp(h, W0, b0.reshape(1, -1), W1, b1.reshape(1, -1),
                W2, b2.reshape(1, -1))


# speedup vs baseline: 8.7433x; 1.3686x over previous
import functools
import numpy as np
import jax
import jax.numpy as jnp
from jax import lax
from jax.experimental import pallas as pl
from jax.experimental.pallas import tpu as pltpu
from jax.experimental.pallas import tpu_sc as plsc

N_LEVELS = 16
F = 2
BASE = 16
TARGET = 2048
LOG2_T = 19
T = 2 ** LOG2_T
DIM = 3
N_PTS = 131072
GRID_OUT = 64
HIDDEN = (GRID_OUT + 15) // 16 * 16
SCALE = np.exp2(np.log2(TARGET / BASE) / (N_LEVELS - 1))
RES = [int(np.floor(BASE * SCALE ** l)) for l in range(N_LEVELS)]
DENSE = [(r + 1) ** 3 <= T for r in RES]
P1, P2 = 2654435761, 805459861
ENC = N_LEVELS * F

NC, NS, LANES = 2, 16, 16
NW = NC * NS
PW = N_PTS // NW
CH = 16
NCHUNK = PW // CH
HBUF = 512
FLUSH_EVERY = HBUF // CH

TBL_ELEMS = N_LEVELS * T * F
RL_CHF = 16384
RL_SPAN = TBL_ELEMS // NW
RL_NCH = RL_SPAN // RL_CHF


def _relayout_body(tin_hbm, tout_hbm, bin_v, bout_v, sem):
    wid = lax.axis_index("s") * NC + lax.axis_index("c")
    base = wid * RL_SPAN
    iota = lax.iota(jnp.int32, LANES)

    def chunk(k, carry):
        off = base + k * RL_CHF
        pltpu.sync_copy(tin_hbm.at[pl.ds(off, RL_CHF)], bin_v)

        def block(b, carry2):
            b256 = b * 256

            def eight(m, carry3):
                m16 = b256 + m * 16
                f0 = bin_v[pl.ds(m16, LANES)]
                f1 = bin_v[pl.ds(m16 + 128, LANES)]
                pos = b256 + m * 32 + 2 * iota
                plsc.store_scatter(bout_v, [pos], f0)
                plsc.store_scatter(bout_v, [pos + 1], f1)
                return carry3

            return lax.fori_loop(0, 8, eight, carry2)

        lax.fori_loop(0, RL_CHF // 256, block, 0)
        pltpu.sync_copy(bout_v, tout_hbm.at[pl.ds(off, RL_CHF)])
        return carry

    lax.fori_loop(0, RL_NCH, chunk, 0)


@jax.jit
def _relayout(tflat):
    mesh = plsc.VectorSubcoreMesh(core_axis_name="c", subcore_axis_name="s",
                                  num_cores=NC, num_subcores=NS)
    return pl.kernel(
        _relayout_body,
        out_type=jax.ShapeDtypeStruct((TBL_ELEMS,), jnp.float32),
        mesh=mesh,
        compiler_params=pltpu.CompilerParams(needs_layout_passes=False,
                                             use_tc_tiling_on_sc=False),
        scratch_types=[
            pltpu.VMEM((RL_CHF,), jnp.float32),
            pltpu.VMEM((RL_CHF,), jnp.float32),
            pltpu.SemaphoreType.DMA,
        ],
    )(tflat)


def _encode_body(xt_hbm, table_hbm, h_hbm, xv, wb, colb, hbuf, sem, *lvl_scratch):
    idxrefs = lvl_scratch[:N_LEVELS]
    rowrefs = lvl_scratch[N_LEVELS:]
    wid = lax.axis_index("s") * NC + lax.axis_index("c")
    base = wid * PW
    pltpu.sync_copy(xt_hbm.at[:, pl.ds(base, PW)], xv)
    iota = lax.iota(jnp.int32, LANES)

    def chunk_body(k, carry):
        off = k * CH
        xc = xv[0, pl.ds(off, CH)]
        yc = xv[1, pl.ds(off, CH)]
        zc = xv[2, pl.ds(off, CH)]

        for l in range(N_LEVELS):
            res = RES[l]
            resf = float(res)
            px = xc * resf
            py = yc * resf
            pz = zc * resf
            ix = px.astype(jnp.int32)
            iy = py.astype(jnp.int32)
            iz = pz.astype(jnp.int32)
            fx = px - ix.astype(jnp.float32)
            fy = py - iy.astype(jnp.float32)
            fz = pz - iz.astype(jnp.float32)
            x0, x1 = ix, jnp.minimum(ix + 1, res)
            y0, y1 = iy, jnp.minimum(iy + 1, res)
            z0, z1 = iz, jnp.minimum(iz + 1, res)
            wx0, wx1 = 1.0 - fx, fx
            wy0, wy1 = 1.0 - fy, fy
            wz0, wz1 = 1.0 - fz, fz
            if not DENSE[l]:
                hx0 = x0.astype(jnp.uint32)
                hx1 = x1.astype(jnp.uint32)
                hy0 = y0.astype(jnp.uint32) * jnp.uint32(P1)
                hy1 = y1.astype(jnp.uint32) * jnp.uint32(P1)
                hz0 = z0.astype(jnp.uint32) * jnp.uint32(P2)
                hz1 = z1.astype(jnp.uint32) * jnp.uint32(P2)
            for c in range(8):
                cxb, cyb, czb = c & 1, (c >> 1) & 1, (c >> 2) & 1
                w = ((wx1 if cxb else wx0) * (wy1 if cyb else wy0)) * (wz1 if czb else wz0)
                if DENSE[l]:
                    s = res + 1
                    idx = (x1 if cxb else x0) + (y1 if cyb else y0) * s \
                        + (z1 if czb else z0) * (s * s) + (l * T)
                else:
                    h = (hx1 if cxb else hx0) ^ (hy1 if cyb else hy0) ^ (hz1 if czb else hz0)
                    idx = (h & jnp.uint32(T - 1)).astype(jnp.int32) + (l * T)
                idxrefs[l][pl.ds(c * LANES, LANES)] = lax.shift_right_logical(idx, 2)
                colb[l, pl.ds(c * LANES, LANES)] = (idx & 3) * 2
                wb[l, pl.ds(c * LANES, LANES)] = w

        copies = []
        for l in range(N_LEVELS):
            cp = pltpu.make_async_copy(table_hbm.at[idxrefs[l]], rowrefs[l], sem)
            cp.start()
            copies.append(cp)

        hcol = (k % FLUSH_EVERY) * CH
        for l in range(N_LEVELS):
            copies[l].wait()
            acc0 = jnp.zeros((LANES,), jnp.float32)
            acc1 = jnp.zeros((LANES,), jnp.float32)
            for c in range(8):
                w = wb[l, pl.ds(c * LANES, LANES)]
                colv = colb[l, pl.ds(c * LANES, LANES)]
                rows = c * LANES + iota
                f0 = plsc.load_gather(rowrefs[l], [rows, colv])
                f1 = plsc.load_gather(rowrefs[l], [rows, colv + 1])
                acc0 = acc0 + f0 * w
                acc1 = acc1 + f1 * w
            hbuf[2 * l, pl.ds(hcol, CH)] = acc0
            hbuf[2 * l + 1, pl.ds(hcol, CH)] = acc1

        @pl.when(k % FLUSH_EVERY == FLUSH_EVERY - 1)
        def _():
            ob = pl.multiple_of(base + (k - (FLUSH_EVERY - 1)) * CH, HBUF)
            pltpu.sync_copy(hbuf, h_hbm.at[:, pl.ds(ob, HBUF)])

        return carry

    lax.fori_loop(0, NCHUNK, chunk_body, 0)


@jax.jit
def _encode(xt, table8):
    mesh = plsc.VectorSubcoreMesh(core_axis_name="c", subcore_axis_name="s",
                                  num_cores=NC, num_subcores=NS)
    return pl.kernel(
        _encode_body,
        out_type=jax.ShapeDtypeStruct((ENC, N_PTS), jnp.float32),
        mesh=mesh,
        compiler_params=pltpu.CompilerParams(needs_layout_passes=False,
                                             use_tc_tiling_on_sc=False),
        scratch_types=(
            [
                pltpu.VMEM((DIM, PW), jnp.float32),
                pltpu.VMEM((N_LEVELS, 8 * LANES), jnp.float32),
                pltpu.VMEM((N_LEVELS, 8 * LANES), jnp.int32),
                pltpu.VMEM((ENC, HBUF), jnp.float32),
                pltpu.SemaphoreType.DMA,
            ]
            + [pltpu.VMEM((8 * LANES,), jnp.int32) for _ in range(N_LEVELS)]
            + [pltpu.VMEM((8 * LANES, 8), jnp.float32) for _ in range(N_LEVELS)]
        ),
    )(xt, table8)


BLK = 1024


def _mlp_body(h_ref, w0_ref, b0_ref, w1_ref, b1_ref, w2_ref, b2_ref, o_ref):
    h = h_ref[...]
    a = lax.dot_general(h, w0_ref[...], (((0,), (0,)), ((), ())),
                        preferred_element_type=jnp.float32,
                        precision=lax.Precision.HIGHEST)
    a = jnp.maximum(a + b0_ref[...], 0.0)
    a = jnp.dot(a, w1_ref[...], preferred_element_type=jnp.float32,
                precision=lax.Precision.HIGHEST)
    a = jnp.maximum(a + b1_ref[...], 0.0)
    o_ref[...] = lax.dot_general(w2_ref[...], a, (((0,), (1,)), ((), ())),
                                 preferred_element_type=jnp.float32,
                                 precision=lax.Precision.HIGHEST) + b2_ref[...]


@jax.jit
def _mlp(h, W0, b0, W1, b1, W2, b2):
    grid = (N_PTS // BLK,)
    return pl.pallas_call(
        _mlp_body,
        grid=grid,
        in_specs=[
            pl.BlockSpec((ENC, BLK), lambda i: (0, i)),
            pl.BlockSpec((ENC, HIDDEN), lambda i: (0, 0)),
            pl.BlockSpec((1, HIDDEN), lambda i: (0, 0)),
            pl.BlockSpec((HIDDEN, HIDDEN), lambda i: (0, 0)),
            pl.BlockSpec((1, HIDDEN), lambda i: (0, 0)),
            pl.BlockSpec((HIDDEN, GRID_OUT), lambda i: (0, 0)),
            pl.BlockSpec((GRID_OUT, 1), lambda i: (0, 0)),
        ],
        out_specs=pl.BlockSpec((GRID_OUT, BLK), lambda i: (0, i)),
        out_shape=jax.ShapeDtypeStruct((GRID_OUT, N_PTS), jnp.float32),
    )(h, W0, b0, W1, b1, W2, b2)


def kernel(x, table, W0, b0, W1, b1, W2, b2):
    xt = x.T
    tphys = (table.transpose(0, 2, 1)
             .reshape(N_LEVELS, F, T // 128, 128)
             .transpose(0, 2, 1, 3)
             .reshape(TBL_ELEMS))
    t8 = _relayout(tphys).reshape(TBL_ELEMS // 8, 8)
    h = _encode(xt, t8)
    out_t = _mlp(h, W0, b0.reshape(1, -1), W1, b1.reshape(1, -1),
                 W2, b2.reshape(-1, 1))
    return out_t.T


# speedup vs baseline: 10.9498x; 1.2524x over previous
import functools
import numpy as np
import jax
import jax.numpy as jnp
from jax import lax
from jax.experimental import pallas as pl
from jax.experimental.pallas import tpu as pltpu
from jax.experimental.pallas import tpu_sc as plsc

N_LEVELS = 16
F = 2
BASE = 16
TARGET = 2048
LOG2_T = 19
T = 2 ** LOG2_T
DIM = 3
N_PTS = 131072
GRID_OUT = 64
HIDDEN = (GRID_OUT + 15) // 16 * 16
SCALE = np.exp2(np.log2(TARGET / BASE) / (N_LEVELS - 1))
RES = [int(np.floor(BASE * SCALE ** l)) for l in range(N_LEVELS)]
DENSE = [(r + 1) ** 3 <= T for r in RES]
P1, P2 = 2654435761, 805459861
ENC = N_LEVELS * F

NC, NS, LANES = 2, 16, 16
NW = NC * NS
PW = N_PTS // NW
CH = 16
NCHUNK = PW // CH
HBUF = 512
FLUSH_EVERY = HBUF // CH

TBL_ELEMS = N_LEVELS * T * F
RL_CHF = 16384
RL_SPAN = TBL_ELEMS // NW
RL_NCH = RL_SPAN // RL_CHF


def _relayout_body(tin_hbm, tout_hbm, bin_v, bout_v, sem):
    wid = lax.axis_index("s") * NC + lax.axis_index("c")
    base = wid * RL_SPAN
    iota = lax.iota(jnp.int32, LANES)

    def chunk(k, carry):
        off = base + k * RL_CHF
        pltpu.sync_copy(tin_hbm.at[pl.ds(off, RL_CHF)], bin_v)

        def block(b, carry2):
            b256 = b * 256

            def eight(m, carry3):
                m16 = b256 + m * 16
                f0 = bin_v[pl.ds(m16, LANES)]
                f1 = bin_v[pl.ds(m16 + 128, LANES)]
                pos = b256 + m * 32 + 2 * iota
                plsc.store_scatter(bout_v, [pos], f0)
                plsc.store_scatter(bout_v, [pos + 1], f1)
                return carry3

            return lax.fori_loop(0, 8, eight, carry2)

        lax.fori_loop(0, RL_CHF // 256, block, 0)
        pltpu.sync_copy(bout_v, tout_hbm.at[pl.ds(off, RL_CHF)])
        return carry

    lax.fori_loop(0, RL_NCH, chunk, 0)


@jax.jit
def _relayout(tflat):
    mesh = plsc.VectorSubcoreMesh(core_axis_name="c", subcore_axis_name="s",
                                  num_cores=NC, num_subcores=NS)
    return pl.kernel(
        _relayout_body,
        out_type=jax.ShapeDtypeStruct((TBL_ELEMS,), jnp.float32),
        mesh=mesh,
        compiler_params=pltpu.CompilerParams(needs_layout_passes=False,
                                             use_tc_tiling_on_sc=False),
        scratch_types=[
            pltpu.VMEM((RL_CHF,), jnp.float32),
            pltpu.VMEM((RL_CHF,), jnp.float32),
            pltpu.SemaphoreType.DMA,
        ],
    )(tflat)


N_STAGED = 2
STAGED_ROWS = [(RES[l] + 1) ** 3 * F // 8 + 8 for l in range(N_STAGED)]


def _encode_body(xt_hbm, table_hbm, h_hbm, xv, wb, colb, hbuf, sem, *lvl_scratch):
    idxrefs = lvl_scratch[:N_LEVELS]
    rowrefs = lvl_scratch[N_LEVELS:2 * N_LEVELS - N_STAGED]
    strefs = lvl_scratch[2 * N_LEVELS - N_STAGED:]
    wid = lax.axis_index("s") * NC + lax.axis_index("c")
    base = wid * PW
    pltpu.sync_copy(xt_hbm.at[:, pl.ds(base, PW)], xv)
    for l in range(N_STAGED):
        pltpu.sync_copy(table_hbm.at[pl.ds(l * (T * F // 8), STAGED_ROWS[l])],
                        strefs[l])
    iota = lax.iota(jnp.int32, LANES)

    def chunk_body(k, carry):
        off = k * CH
        xc = xv[0, pl.ds(off, CH)]
        yc = xv[1, pl.ds(off, CH)]
        zc = xv[2, pl.ds(off, CH)]

        for l in range(N_LEVELS):
            res = RES[l]
            resf = float(res)
            px = xc * resf
            py = yc * resf
            pz = zc * resf
            ix = px.astype(jnp.int32)
            iy = py.astype(jnp.int32)
            iz = pz.astype(jnp.int32)
            fx = px - ix.astype(jnp.float32)
            fy = py - iy.astype(jnp.float32)
            fz = pz - iz.astype(jnp.float32)
            x0, x1 = ix, jnp.minimum(ix + 1, res)
            y0, y1 = iy, jnp.minimum(iy + 1, res)
            z0, z1 = iz, jnp.minimum(iz + 1, res)
            wx0, wx1 = 1.0 - fx, fx
            wy0, wy1 = 1.0 - fy, fy
            wz0, wz1 = 1.0 - fz, fz
            if not DENSE[l]:
                hx0 = x0.astype(jnp.uint32)
                hx1 = x1.astype(jnp.uint32)
                hy0 = y0.astype(jnp.uint32) * jnp.uint32(P1)
                hy1 = y1.astype(jnp.uint32) * jnp.uint32(P1)
                hz0 = z0.astype(jnp.uint32) * jnp.uint32(P2)
                hz1 = z1.astype(jnp.uint32) * jnp.uint32(P2)
            for c in range(8):
                cxb, cyb, czb = c & 1, (c >> 1) & 1, (c >> 2) & 1
                w = ((wx1 if cxb else wx0) * (wy1 if cyb else wy0)) * (wz1 if czb else wz0)
                if DENSE[l]:
                    s = res + 1
                    idx = (x1 if cxb else x0) + (y1 if cyb else y0) * s \
                        + (z1 if czb else z0) * (s * s)
                    if l >= N_STAGED:
                        idx = idx + (l * T)
                else:
                    h = (hx1 if cxb else hx0) ^ (hy1 if cyb else hy0) ^ (hz1 if czb else hz0)
                    idx = (h & jnp.uint32(T - 1)).astype(jnp.int32) + (l * T)
                idxrefs[l][pl.ds(c * LANES, LANES)] = lax.shift_right_logical(idx, 2)
                colb[l, pl.ds(c * LANES, LANES)] = (idx & 3) * 2
                wb[l, pl.ds(c * LANES, LANES)] = w

        copies = {}
        for l in range(N_STAGED, N_LEVELS):
            cp = pltpu.make_async_copy(table_hbm.at[idxrefs[l]],
                                       rowrefs[l - N_STAGED], sem)
            cp.start()
            copies[l] = cp

        hcol = (k % FLUSH_EVERY) * CH
        for l in range(N_LEVELS):
            acc0 = jnp.zeros((LANES,), jnp.float32)
            acc1 = jnp.zeros((LANES,), jnp.float32)
            if l < N_STAGED:
                src = strefs[l]
            else:
                copies[l].wait()
                src = rowrefs[l - N_STAGED]
            for c in range(8):
                w = wb[l, pl.ds(c * LANES, LANES)]
                colv = colb[l, pl.ds(c * LANES, LANES)]
                if l < N_STAGED:
                    rows = idxrefs[l][pl.ds(c * LANES, LANES)]
                else:
                    rows = c * LANES + iota
                f0 = plsc.load_gather(src, [rows, colv])
                f1 = plsc.load_gather(src, [rows, colv + 1])
                acc0 = acc0 + f0 * w
                acc1 = acc1 + f1 * w
            hbuf[2 * l, pl.ds(hcol, CH)] = acc0
            hbuf[2 * l + 1, pl.ds(hcol, CH)] = acc1

        @pl.when(k % FLUSH_EVERY == FLUSH_EVERY - 1)
        def _():
            ob = pl.multiple_of(base + (k - (FLUSH_EVERY - 1)) * CH, HBUF)
            pltpu.sync_copy(hbuf, h_hbm.at[:, pl.ds(ob, HBUF)])

        return carry

    lax.fori_loop(0, NCHUNK, chunk_body, 0)


@jax.jit
def _encode(xt, table8):
    mesh = plsc.VectorSubcoreMesh(core_axis_name="c", subcore_axis_name="s",
                                  num_cores=NC, num_subcores=NS)
    return pl.kernel(
        _encode_body,
        out_type=jax.ShapeDtypeStruct((ENC, N_PTS), jnp.float32),
        mesh=mesh,
        compiler_params=pltpu.CompilerParams(needs_layout_passes=False,
                                             use_tc_tiling_on_sc=False),
        scratch_types=(
            [
                pltpu.VMEM((DIM, PW), jnp.float32),
                pltpu.VMEM((N_LEVELS, 8 * LANES), jnp.float32),
                pltpu.VMEM((N_LEVELS, 8 * LANES), jnp.int32),
                pltpu.VMEM((ENC, HBUF), jnp.float32),
                pltpu.SemaphoreType.DMA,
            ]
            + [pltpu.VMEM((8 * LANES,), jnp.int32) for _ in range(N_LEVELS)]
            + [pltpu.VMEM((8 * LANES, 8), jnp.float32)
               for _ in range(N_LEVELS - N_STAGED)]
            + [pltpu.VMEM((STAGED_ROWS[l], 8), jnp.float32)
               for l in range(N_STAGED)]
        ),
    )(xt, table8)


BLK = 1024


def _mlp_body(h_ref, w0_ref, b0_ref, w1_ref, b1_ref, w2_ref, b2_ref, o_ref):
    h = h_ref[...]
    a = lax.dot_general(h, w0_ref[...], (((0,), (0,)), ((), ())),
                        preferred_element_type=jnp.float32)
    a = jnp.maximum(a + b0_ref[...], 0.0)
    a = jnp.dot(a, w1_ref[...], preferred_element_type=jnp.float32)
    a = jnp.maximum(a + b1_ref[...], 0.0)
    o_ref[...] = lax.dot_general(w2_ref[...], a, (((0,), (1,)), ((), ())),
                                 preferred_element_type=jnp.float32) + b2_ref[...]


@jax.jit
def _mlp(h, W0, b0, W1, b1, W2, b2):
    grid = (N_PTS // BLK,)
    return pl.pallas_call(
        _mlp_body,
        grid=grid,
        in_specs=[
            pl.BlockSpec((ENC, BLK), lambda i: (0, i)),
            pl.BlockSpec((ENC, HIDDEN), lambda i: (0, 0)),
            pl.BlockSpec((1, HIDDEN), lambda i: (0, 0)),
            pl.BlockSpec((HIDDEN, HIDDEN), lambda i: (0, 0)),
            pl.BlockSpec((1, HIDDEN), lambda i: (0, 0)),
            pl.BlockSpec((HIDDEN, GRID_OUT), lambda i: (0, 0)),
            pl.BlockSpec((GRID_OUT, 1), lambda i: (0, 0)),
        ],
        out_specs=pl.BlockSpec((GRID_OUT, BLK), lambda i: (0, i)),
        out_shape=jax.ShapeDtypeStruct((GRID_OUT, N_PTS), jnp.float32),
    )(h, W0, b0, W1, b1, W2, b2)


def kernel(x, table, W0, b0, W1, b1, W2, b2):
    xt = x.T
    tphys = (table.transpose(0, 2, 1)
             .reshape(N_LEVELS, F, T // 128, 128)
             .transpose(0, 2, 1, 3)
             .reshape(TBL_ELEMS))
    t8 = _relayout(tphys).reshape(TBL_ELEMS // 8, 8)
    h = _encode(xt, t8)
    out_t = _mlp(h, W0, b0.reshape(1, -1), W1, b1.reshape(1, -1),
                 W2, b2.reshape(-1, 1))
    return out_t.T
